# Initial kernel scaffold; baseline (speedup 1.0000x reference)
import jax
import jax.numpy as jnp
from jax.experimental import pallas as pl


def kernel(coords, feats, W1, W2):
    raise NotImplementedError("write your pallas kernel here")


# speedup vs baseline: 11.4132x; 11.4132x over previous
import functools

import jax
import jax.numpy as jnp
from jax import lax
from jax.experimental import pallas as pl
from jax.experimental.pallas import tpu as pltpu
from jax.experimental.pallas import tpu_sc as plsc

B = 4
IMG_H = 512
IMG_W = 512
N = 200000
IN_CH = 3
WIDTH = 64
HW = IMG_H * IMG_W
SZ = B * HW

NC = 2
NS = 16
NW = NC * NS
LANES = 16

NPAD = 204800
PER_TILE = NPAD // NW

IMG_CH = 4096
IMG_REPS = 17
IMGW = NS * IMG_REPS * IMG_CH
JUNK_SLOT = SZ
EMPTY_SLOT = SZ + 8

OFFSETS = tuple((dy, dx) for dy in (-1, 0, 1) for dx in (-1, 0, 1))

_mesh = plsc.VectorSubcoreMesh(core_axis_name="c", subcore_axis_name="s")
_cp = pltpu.CompilerParams(use_tc_tiling_on_sc=False, needs_layout_passes=False)


def _worker_id():
    return lax.axis_index("s") * NC + lax.axis_index("c")


P1_CW = 1600


@functools.partial(
    pl.kernel,
    out_type=jax.ShapeDtypeStruct((IMGW, 16), jnp.float32),
    mesh=_mesh,
    compiler_params=_cp,
    scratch_types=[
        pltpu.VMEM((IMG_CH, 16), jnp.float32),
        pltpu.VMEM((P1_CW,), jnp.int32),
        pltpu.VMEM((P1_CW,), jnp.int32),
        pltpu.VMEM((P1_CW, 16), jnp.float32),
        pltpu.VMEM((P1_CW, 16), jnp.float32),
        pltpu.SemaphoreType.DMA,
        pltpu.SemaphoreType.DMA,
        pltpu.SemaphoreType.DMA,
    ],
)
def _p1_build_img(pat_hbm, flat_hbm, rows_hbm, img_hbm, fillb, addr0, addr1,
                  src0, src1, fsem, isem, ssem):
    addrb = (addr0, addr1)
    srcb = (src0, src1)
    c = lax.axis_index("c")
    s = lax.axis_index("s")

    @pl.when(c == 0)
    def _():
        pltpu.async_copy(pat_hbm, fillb, fsem).wait()
        for r in range(IMG_REPS):
            pltpu.async_copy(
                fillb, img_hbm.at[pl.ds((s * IMG_REPS + r) * IMG_CH, IMG_CH)],
                fsem)
        for r in range(IMG_REPS):
            pltpu.make_async_copy(
                fillb, img_hbm.at[pl.ds(s * IMG_CH, IMG_CH)], fsem).wait()

    plsc.subcore_barrier()

    @pl.when((c == 0) & (s == 0))
    def _():
        per = NPAD
        nch = per // P1_CW

        def load(t, p):
            base = t * P1_CW
            pltpu.async_copy(flat_hbm.at[pl.ds(base, P1_CW)], addrb[p],
                             isem)
            pltpu.async_copy(rows_hbm.at[pl.ds(base, P1_CW)], srcb[p],
                             isem)

        def wait_load(p):
            pltpu.make_async_copy(flat_hbm.at[pl.ds(0, P1_CW)], addrb[p],
                                  isem).wait()
            pltpu.make_async_copy(rows_hbm.at[pl.ds(0, P1_CW)], srcb[p],
                                  isem).wait()

        def wait_scat(p):
            pltpu.make_async_copy(srcb[p], img_hbm.at[addrb[p]],
                                  ssem).wait()

        @pl.loop(0, nch)
        def _(t):
            load(t, 0)
            wait_load(0)
            pltpu.async_copy(srcb[0], img_hbm.at[addrb[0]], ssem)
            wait_scat(0)


P2_CW = 320
P2_NCH = PER_TILE // P2_CW


@functools.partial(
    pl.kernel,
    out_type=jax.ShapeDtypeStruct((9, NPAD, 16), jnp.float32),
    mesh=_mesh,
    compiler_params=_cp,
    scratch_types=(
        [pltpu.VMEM((2, P2_CW), jnp.int32) for _ in range(2)]
        + [pltpu.VMEM((P2_CW,), jnp.int32) for _ in range(18)]
        + [pltpu.VMEM((P2_CW, 16), jnp.float32) for _ in range(18)]
        + [pltpu.SemaphoreType.DMA, pltpu.SemaphoreType.DMA,
           pltpu.SemaphoreType.DMA, pltpu.SemaphoreType.DMA]
    ),
)
def _p2_gather(flat_hbm, vmask_hbm, img_hbm, g1_hbm, *scr):
    fb, mb = scr[0:2]
    _nf = scr[2:20]
    nfb = [( _nf[2 * k], _nf[2 * k + 1]) for k in range(9)]
    _gb = scr[20:38]
    gb = [(_gb[2 * k], _gb[2 * k + 1]) for k in range(9)]
    isem, gsem0, gsem1, wsem = scr[38:42]
    gsem = (gsem0, gsem1)
    wid = _worker_id()
    base0 = wid * PER_TILE

    def load_inputs(t, p):
        base = base0 + t * P2_CW
        pltpu.async_copy(flat_hbm.at[pl.ds(base, P2_CW)], fb.at[p], isem)
        pltpu.async_copy(vmask_hbm.at[pl.ds(base, P2_CW)], mb.at[p], isem)

    def wait_inputs(p):
        pltpu.make_async_copy(flat_hbm.at[pl.ds(0, P2_CW)], fb.at[p],
                              isem).wait()
        pltpu.make_async_copy(vmask_hbm.at[pl.ds(0, P2_CW)], mb.at[p],
                              isem).wait()

    def compute_nflat(p):
        for k, (dy, dx) in enumerate(OFFSETS):
            d = dy * IMG_W + dx

            @pl.loop(0, P2_CW, step=LANES)
            def _(j, _d=d, _k=k):
                f16 = fb[p, pl.ds(j, LANES)]
                m16 = mb[p, pl.ds(j, LANES)]
                ok = ((m16 >> _k) & 1) == 1
                nfb[_k][p][pl.ds(j, LANES)] = jnp.where(ok, f16 + _d,
                                                        EMPTY_SLOT)

    def fire_gathers(p):
        for k in range(9):
            pltpu.async_copy(img_hbm.at[nfb[k][p]], gb[k][p], gsem[p])

    def wait_gathers(p):
        for k in range(9):
            pltpu.make_async_copy(img_hbm.at[nfb[k][p]], gb[k][p],
                                  gsem[p]).wait()

    def fire_writes(t, p):
        base = base0 + t * P2_CW
        for k in range(9):
            pltpu.async_copy(gb[k][p],
                             g1_hbm.at[k].at[pl.ds(base, P2_CW)], wsem)

    def wait_writes(p):
        for k in range(9):
            pltpu.make_async_copy(gb[k][p],
                                  g1_hbm.at[k].at[pl.ds(0, P2_CW)],
                                  wsem).wait()

    @pl.loop(0, P2_NCH)
    def _(t):
        load_inputs(t, 0)
        wait_inputs(0)
        compute_nflat(0)
        fire_gathers(0)
        wait_gathers(0)
        fire_writes(t, 0)
        wait_writes(0)


P3_CT = 2048


def _p3_body(g1_ref, w1_ref, w2_ref, h_ref):
    acc = jnp.zeros((P3_CT, WIDTH), jnp.float32)
    for k in range(9):
        acc = acc + jnp.dot(g1_ref[k], w1_ref[k],
                            precision=lax.Precision.HIGHEST,
                            preferred_element_type=jnp.float32)
    h = jnp.maximum(acc, 0.0)
    for k in range(9):
        h_ref[k] = jnp.dot(h, w2_ref[k], precision=lax.Precision.HIGHEST,
                           preferred_element_type=jnp.float32)


_p3_call = pl.pallas_call(
    _p3_body,
    grid=(NPAD // P3_CT,),
    in_specs=[
        pl.BlockSpec((9, P3_CT, 16), lambda i: (0, i, 0)),
        pl.BlockSpec((9, 16, WIDTH), lambda i: (0, 0, 0)),
        pl.BlockSpec((9, WIDTH, WIDTH), lambda i: (0, 0, 0)),
    ],
    out_specs=pl.BlockSpec((9, P3_CT, WIDTH), lambda i: (0, i, 0)),
    out_shape=jax.ShapeDtypeStruct((9, NPAD, WIDTH), jnp.float32),
)


P4_CW = 80
P4_NCH = PER_TILE // P4_CW


@functools.partial(
    pl.kernel,
    out_type=(
        jax.ShapeDtypeStruct((NW, 16, WIDTH), jnp.float32),
        jax.ShapeDtypeStruct((NW, 16), jnp.int32),
    ),
    mesh=_mesh,
    compiler_params=_cp,
    scratch_types=(
        [pltpu.VMEM((P4_CW, 16), jnp.float32) for _ in range(18)]
        + [pltpu.VMEM((P4_CW,), jnp.int32) for _ in range(18)]
        + [pltpu.VMEM((P4_CW, WIDTH), jnp.float32) for _ in range(18)]
        + [
            pltpu.VMEM((2, P4_CW), jnp.int32),
            pltpu.VMEM((16, WIDTH), jnp.float32),
            pltpu.VMEM((16,), jnp.int32),
            pltpu.SemaphoreType.DMA,
            pltpu.SemaphoreType.DMA,
            pltpu.SemaphoreType.DMA,
        ]
    ),
)
def _p4_reduce(h_hbm, g1_hbm, cb_hbm, sums_hbm, cnt_hbm, *scr):
    _gg = scr[0:18]
    gg = [(_gg[2 * k], _gg[2 * k + 1]) for k in range(9)]
    _ib = scr[18:36]
    idxb = [(_ib[2 * k], _ib[2 * k + 1]) for k in range(9)]
    _hb = scr[36:54]
    hb = [(_hb[2 * k], _hb[2 * k + 1]) for k in range(9)]
    bb, accb, cntb, isem, gsem0, gsem1 = scr[54:60]
    gsem = (gsem0, gsem1)
    wid = _worker_id()
    base0 = wid * PER_TILE

    for r in range(16):
        for cg in range(WIDTH // LANES):
            accb[r, pl.ds(cg * LANES, LANES)] = jnp.zeros((LANES,),
                                                          jnp.float32)
    cntb[...] = jnp.zeros((16,), jnp.int32)

    def load_rows(t, p):
        base = base0 + t * P4_CW
        for k in range(9):
            pltpu.async_copy(g1_hbm.at[k].at[pl.ds(base, P4_CW)],
                             gg[k][p], isem)
        pltpu.async_copy(cb_hbm.at[pl.ds(base, P4_CW)], bb.at[p], isem)

    def wait_rows(p):
        for k in range(9):
            pltpu.make_async_copy(g1_hbm.at[k].at[pl.ds(0, P4_CW)],
                                  gg[k][p], isem).wait()
        pltpu.make_async_copy(cb_hbm.at[pl.ds(0, P4_CW)], bb.at[p],
                              isem).wait()

    def extract_idx(p):
        rid = lax.iota(jnp.int32, LANES)
        col4 = jnp.full((LANES,), 4, jnp.int32)
        for k in range(9):
            for grp in range(P4_CW // LANES):
                v = plsc.load_gather(gg[k][p],
                                     [rid + grp * LANES, col4])
                idxb[k][p][pl.ds(grp * LANES, LANES)] = (
                    v.astype(jnp.int32) + k * NPAD)

    def fire_gathers(p):
        for k in range(9):
            pltpu.async_copy(h_hbm.at[idxb[k][p]], hb[k][p], gsem[p])

    def wait_gathers(p):
        for k in range(9):
            pltpu.make_async_copy(h_hbm.at[idxb[k][p]], hb[k][p],
                                  gsem[p]).wait()

    def compute(p):
        @pl.loop(0, P4_CW, step=LANES)
        def _(j):
            b16 = bb[p, pl.ds(j, LANES)]
            plsc.addupdate_scatter(cntb, [b16],
                                   jnp.full((LANES,), 1, jnp.int32))
            for ri in range(LANES):
                brow = b16[ri]
                r = j + ri
                for cg in range(WIDTH // LANES):
                    sl = pl.ds(cg * LANES, LANES)
                    v01 = hb[0][p][r, sl] + hb[1][p][r, sl]
                    v23 = hb[2][p][r, sl] + hb[3][p][r, sl]
                    v45 = hb[4][p][r, sl] + hb[5][p][r, sl]
                    v67 = hb[6][p][r, sl] + hb[7][p][r, sl]
                    v = ((v01 + v23) + (v45 + v67)) + hb[8][p][r, sl]
                    v = jnp.maximum(v, 0.0)
                    accb[brow, sl] = accb[brow, sl] + v

    @pl.loop(0, P4_NCH)
    def _(t):
        load_rows(t, 0)
        wait_rows(0)
        extract_idx(0)
        fire_gathers(0)
        wait_gathers(0)
        compute(0)

    pltpu.sync_copy(accb, sums_hbm.at[wid])
    pltpu.sync_copy(cntb, cnt_hbm.at[wid])


def kernel(coords, feats, W1, W2):
    ar = jnp.arange(NPAD, dtype=jnp.int32)
    live = ar < N

    cb = jnp.zeros((NPAD,), jnp.int32).at[:N].set(coords[:, 0])
    cy = jnp.zeros((NPAD,), jnp.int32).at[:N].set(coords[:, 1])
    cx = jnp.zeros((NPAD,), jnp.int32).at[:N].set(coords[:, 2])
    flat = cb * HW + cy * IMG_W + cx
    flat = jnp.where(live, flat, JUNK_SLOT)
    cb_pool = jnp.where(live, cb, 8)

    vmask = jnp.zeros((NPAD,), jnp.int32)
    for k, (dy, dx) in enumerate(OFFSETS):
        ok = ((cy + dy >= 0) & (cy + dy < IMG_H)
              & (cx + dx >= 0) & (cx + dx < IMG_W) & live)
        vmask = vmask | (ok.astype(jnp.int32) << k)

    rows = jnp.zeros((NPAD, 16), jnp.float32)
    rows = rows.at[:N, :IN_CH].set(feats)
    rows = rows.at[:, 4].set(ar.astype(jnp.float32))

    w1p = jnp.zeros((9, 16, WIDTH), jnp.float32).at[:, :IN_CH, :].set(W1)

    pat = jnp.zeros((IMG_CH, 16), jnp.float32).at[:, 4].set(float(N))

    cimg = _p1_build_img(pat, flat, rows)
    g1 = _p2_gather(flat, vmask, cimg)
    h_slabs = _p3_call(g1, w1p, W2)
    sums, cnts = _p4_reduce(h_slabs.reshape(9 * NPAD, WIDTH), g1, cb_pool)

    tot = sums[:, :B, :].sum(axis=0)
    cnt = cnts[:, :B].sum(axis=0).astype(jnp.float32)
    return tot / jnp.maximum(cnt, 1.0)[:, None]


# speedup vs baseline: 11.7851x; 1.0326x over previous
import functools

import jax
import jax.numpy as jnp
from jax import lax
from jax.experimental import pallas as pl
from jax.experimental.pallas import tpu as pltpu
from jax.experimental.pallas import tpu_sc as plsc

B = 4
IMG_H = 512
IMG_W = 512
N = 200000
IN_CH = 3
WIDTH = 64
HW = IMG_H * IMG_W
SZ = B * HW

NC = 2
NS = 16
NW = NC * NS
LANES = 16

NPAD = 204800
PER_TILE = NPAD // NW

IMG_CH = 4096
IMG_REPS = 17
IMGW = NS * IMG_REPS * IMG_CH
JUNK_SLOT = SZ
EMPTY_SLOT = SZ + 8

OFFSETS = tuple((dy, dx) for dy in (-1, 0, 1) for dx in (-1, 0, 1))

_mesh = plsc.VectorSubcoreMesh(core_axis_name="c", subcore_axis_name="s")
_cp = pltpu.CompilerParams(use_tc_tiling_on_sc=False, needs_layout_passes=False)


def _worker_id():
    return lax.axis_index("s") * NC + lax.axis_index("c")


P1_CW = 1600


@functools.partial(
    pl.kernel,
    out_type=jax.ShapeDtypeStruct((IMGW, 16), jnp.float32),
    mesh=_mesh,
    compiler_params=_cp,
    scratch_types=[
        pltpu.VMEM((IMG_CH, 16), jnp.float32),
        pltpu.VMEM((P1_CW,), jnp.int32),
        pltpu.VMEM((P1_CW,), jnp.int32),
        pltpu.VMEM((P1_CW, 16), jnp.float32),
        pltpu.VMEM((P1_CW, 16), jnp.float32),
        pltpu.SemaphoreType.DMA,
        pltpu.SemaphoreType.DMA,
        pltpu.SemaphoreType.DMA,
    ],
)
def _p1_build_img(pat_hbm, flat_hbm, rows_hbm, img_hbm, fillb, addr0, addr1,
                  src0, src1, fsem, isem, ssem):
    addrb = (addr0, addr1)
    srcb = (src0, src1)
    c = lax.axis_index("c")
    s = lax.axis_index("s")

    @pl.when(c == 0)
    def _():
        pltpu.async_copy(pat_hbm, fillb, fsem).wait()
        for r in range(IMG_REPS):
            pltpu.async_copy(
                fillb, img_hbm.at[pl.ds((s * IMG_REPS + r) * IMG_CH, IMG_CH)],
                fsem)
        for r in range(IMG_REPS):
            pltpu.make_async_copy(
                fillb, img_hbm.at[pl.ds(s * IMG_CH, IMG_CH)], fsem).wait()

    plsc.subcore_barrier()

    @pl.when(c == 0)
    def _():
        per = NPAD // NS
        nch = per // P1_CW

        def load(t, p):
            base = s * per + t * P1_CW
            pltpu.async_copy(flat_hbm.at[pl.ds(base, P1_CW)], addrb[p],
                             isem)
            pltpu.async_copy(rows_hbm.at[pl.ds(base, P1_CW)], srcb[p],
                             isem)

        def wait_load(p):
            pltpu.make_async_copy(flat_hbm.at[pl.ds(0, P1_CW)], addrb[p],
                                  isem).wait()
            pltpu.make_async_copy(rows_hbm.at[pl.ds(0, P1_CW)], srcb[p],
                                  isem).wait()

        def wait_scat(p):
            pltpu.make_async_copy(srcb[p], img_hbm.at[addrb[p]],
                                  ssem).wait()

        @pl.loop(0, nch)
        def _(t):
            load(t, 0)
            wait_load(0)
            pltpu.async_copy(srcb[0], img_hbm.at[addrb[0]], ssem)
            wait_scat(0)


P2_CW = 640
P2_NCH = PER_TILE // P2_CW


@functools.partial(
    pl.kernel,
    out_type=jax.ShapeDtypeStruct((9, NPAD, 16), jnp.float32),
    mesh=_mesh,
    compiler_params=_cp,
    scratch_types=(
        [pltpu.VMEM((2, P2_CW), jnp.int32) for _ in range(2)]
        + [pltpu.VMEM((P2_CW,), jnp.int32) for _ in range(9)]
        + [pltpu.VMEM((P2_CW, 16), jnp.float32) for _ in range(9)]
        + [pltpu.SemaphoreType.DMA, pltpu.SemaphoreType.DMA,
           pltpu.SemaphoreType.DMA, pltpu.SemaphoreType.DMA]
    ),
)
def _p2_gather(flat_hbm, vmask_hbm, img_hbm, g1_hbm, *scr):
    fb, mb = scr[0:2]
    _nf = scr[2:11]
    nfb = [(_nf[k], _nf[k]) for k in range(9)]
    _gb = scr[11:20]
    gb = [(_gb[k], _gb[k]) for k in range(9)]
    isem, gsem0, gsem1, wsem = scr[20:24]
    gsem = (gsem0, gsem1)
    wid = _worker_id()
    base0 = wid * PER_TILE

    def load_inputs(t, p):
        base = base0 + t * P2_CW
        pltpu.async_copy(flat_hbm.at[pl.ds(base, P2_CW)], fb.at[p], isem)
        pltpu.async_copy(vmask_hbm.at[pl.ds(base, P2_CW)], mb.at[p], isem)

    def wait_inputs(p):
        pltpu.make_async_copy(flat_hbm.at[pl.ds(0, P2_CW)], fb.at[p],
                              isem).wait()
        pltpu.make_async_copy(vmask_hbm.at[pl.ds(0, P2_CW)], mb.at[p],
                              isem).wait()

    def compute_nflat(p):
        for k, (dy, dx) in enumerate(OFFSETS):
            d = dy * IMG_W + dx

            @pl.loop(0, P2_CW, step=LANES)
            def _(j, _d=d, _k=k):
                f16 = fb[p, pl.ds(j, LANES)]
                m16 = mb[p, pl.ds(j, LANES)]
                ok = ((m16 >> _k) & 1) == 1
                nfb[_k][p][pl.ds(j, LANES)] = jnp.where(ok, f16 + _d,
                                                        EMPTY_SLOT)

    def fire_gathers(p):
        for k in range(9):
            pltpu.async_copy(img_hbm.at[nfb[k][p]], gb[k][p], gsem[p])

    def wait_gathers(p):
        for k in range(9):
            pltpu.make_async_copy(img_hbm.at[nfb[k][p]], gb[k][p],
                                  gsem[p]).wait()

    def fire_writes(t, p):
        base = base0 + t * P2_CW
        for k in range(9):
            pltpu.async_copy(gb[k][p],
                             g1_hbm.at[k].at[pl.ds(base, P2_CW)], wsem)

    def wait_writes(p):
        for k in range(9):
            pltpu.make_async_copy(gb[k][p],
                                  g1_hbm.at[k].at[pl.ds(0, P2_CW)],
                                  wsem).wait()

    @pl.loop(0, P2_NCH)
    def _(t):
        load_inputs(t, 0)
        wait_inputs(0)
        compute_nflat(0)
        fire_gathers(0)
        wait_gathers(0)
        fire_writes(t, 0)
        wait_writes(0)


P3_CT = 2048


def _p3_body(g1_ref, w1_ref, w2_ref, h_ref):
    acc = jnp.zeros((P3_CT, WIDTH), jnp.float32)
    for k in range(9):
        acc = acc + jnp.dot(g1_ref[k], w1_ref[k],
                            precision=lax.Precision.HIGHEST,
                            preferred_element_type=jnp.float32)
    h = jnp.maximum(acc, 0.0)
    for k in range(9):
        h_ref[k] = jnp.dot(h, w2_ref[k], precision=lax.Precision.HIGHEST,
                           preferred_element_type=jnp.float32)


_p3_call = pl.pallas_call(
    _p3_body,
    grid=(NPAD // P3_CT,),
    in_specs=[
        pl.BlockSpec((9, P3_CT, 16), lambda i: (0, i, 0)),
        pl.BlockSpec((9, 16, WIDTH), lambda i: (0, 0, 0)),
        pl.BlockSpec((9, WIDTH, WIDTH), lambda i: (0, 0, 0)),
    ],
    out_specs=pl.BlockSpec((9, P3_CT, WIDTH), lambda i: (0, i, 0)),
    out_shape=jax.ShapeDtypeStruct((9, NPAD, WIDTH), jnp.float32),
)


P4_CW = 128
P4_NCH = PER_TILE // P4_CW


@functools.partial(
    pl.kernel,
    out_type=(
        jax.ShapeDtypeStruct((NW, 16, WIDTH), jnp.float32),
        jax.ShapeDtypeStruct((NW, 16), jnp.int32),
    ),
    mesh=_mesh,
    compiler_params=_cp,
    scratch_types=(
        [pltpu.VMEM((P4_CW, 16), jnp.float32) for _ in range(9)]
        + [pltpu.VMEM((P4_CW,), jnp.int32) for _ in range(9)]
        + [pltpu.VMEM((P4_CW, WIDTH), jnp.float32) for _ in range(9)]
        + [
            pltpu.VMEM((2, P4_CW), jnp.int32),
            pltpu.VMEM((16, WIDTH), jnp.float32),
            pltpu.VMEM((16,), jnp.int32),
            pltpu.SemaphoreType.DMA,
            pltpu.SemaphoreType.DMA,
            pltpu.SemaphoreType.DMA,
        ]
    ),
)
def _p4_reduce(h_hbm, g1_hbm, cb_hbm, sums_hbm, cnt_hbm, *scr):
    _gg = scr[0:9]
    gg = [(_gg[k], _gg[k]) for k in range(9)]
    _ib = scr[9:18]
    idxb = [(_ib[k], _ib[k]) for k in range(9)]
    _hb = scr[18:27]
    hb = [(_hb[k], _hb[k]) for k in range(9)]
    bb, accb, cntb, isem, gsem0, gsem1 = scr[27:33]
    gsem = (gsem0, gsem1)
    wid = _worker_id()
    base0 = wid * PER_TILE

    for r in range(16):
        for cg in range(WIDTH // LANES):
            accb[r, pl.ds(cg * LANES, LANES)] = jnp.zeros((LANES,),
                                                          jnp.float32)
    cntb[...] = jnp.zeros((16,), jnp.int32)

    def load_rows(t, p):
        base = base0 + t * P4_CW
        for k in range(9):
            pltpu.async_copy(g1_hbm.at[k].at[pl.ds(base, P4_CW)],
                             gg[k][p], isem)
        pltpu.async_copy(cb_hbm.at[pl.ds(base, P4_CW)], bb.at[p], isem)

    def wait_rows(p):
        for k in range(9):
            pltpu.make_async_copy(g1_hbm.at[k].at[pl.ds(0, P4_CW)],
                                  gg[k][p], isem).wait()
        pltpu.make_async_copy(cb_hbm.at[pl.ds(0, P4_CW)], bb.at[p],
                              isem).wait()

    def extract_idx(p):
        rid = lax.iota(jnp.int32, LANES)
        col4 = jnp.full((LANES,), 4, jnp.int32)
        for k in range(9):
            for grp in range(P4_CW // LANES):
                v = plsc.load_gather(gg[k][p],
                                     [rid + grp * LANES, col4])
                idxb[k][p][pl.ds(grp * LANES, LANES)] = (
                    v.astype(jnp.int32) + k * NPAD)

    def fire_gathers(p):
        for k in range(9):
            pltpu.async_copy(h_hbm.at[idxb[k][p]], hb[k][p], gsem[p])

    def wait_gathers(p):
        for k in range(9):
            pltpu.make_async_copy(h_hbm.at[idxb[k][p]], hb[k][p],
                                  gsem[p]).wait()

    def compute(p):
        @pl.loop(0, P4_CW, step=LANES)
        def _(j):
            b16 = bb[p, pl.ds(j, LANES)]
            plsc.addupdate_scatter(cntb, [b16],
                                   jnp.full((LANES,), 1, jnp.int32))
            for ri in range(LANES):
                brow = b16[ri]
                r = j + ri
                for cg in range(WIDTH // LANES):
                    sl = pl.ds(cg * LANES, LANES)
                    v01 = hb[0][p][r, sl] + hb[1][p][r, sl]
                    v23 = hb[2][p][r, sl] + hb[3][p][r, sl]
                    v45 = hb[4][p][r, sl] + hb[5][p][r, sl]
                    v67 = hb[6][p][r, sl] + hb[7][p][r, sl]
                    v = ((v01 + v23) + (v45 + v67)) + hb[8][p][r, sl]
                    v = jnp.maximum(v, 0.0)
                    accb[brow, sl] = accb[brow, sl] + v

    @pl.loop(0, P4_NCH)
    def _(t):
        load_rows(t, 0)
        wait_rows(0)
        extract_idx(0)
        fire_gathers(0)
        wait_gathers(0)
        compute(0)

    pltpu.sync_copy(accb, sums_hbm.at[wid])
    pltpu.sync_copy(cntb, cnt_hbm.at[wid])


def kernel(coords, feats, W1, W2):
    ar = jnp.arange(NPAD, dtype=jnp.int32)
    live = ar < N

    cb = jnp.zeros((NPAD,), jnp.int32).at[:N].set(coords[:, 0])
    cy = jnp.zeros((NPAD,), jnp.int32).at[:N].set(coords[:, 1])
    cx = jnp.zeros((NPAD,), jnp.int32).at[:N].set(coords[:, 2])
    flat = cb * HW + cy * IMG_W + cx
    flat = jnp.where(live, flat, JUNK_SLOT)
    cb_pool = jnp.where(live, cb, 8)

    vmask = jnp.zeros((NPAD,), jnp.int32)
    for k, (dy, dx) in enumerate(OFFSETS):
        ok = ((cy + dy >= 0) & (cy + dy < IMG_H)
              & (cx + dx >= 0) & (cx + dx < IMG_W) & live)
        vmask = vmask | (ok.astype(jnp.int32) << k)

    rows = jnp.zeros((NPAD, 16), jnp.float32)
    rows = rows.at[:N, :IN_CH].set(feats)
    rows = rows.at[:, 4].set(ar.astype(jnp.float32))

    w1p = jnp.zeros((9, 16, WIDTH), jnp.float32).at[:, :IN_CH, :].set(W1)

    pat = jnp.zeros((IMG_CH, 16), jnp.float32).at[:, 4].set(float(N))

    cimg = _p1_build_img(pat, flat, rows)
    g1 = _p2_gather(flat, vmask, cimg)
    h_slabs = _p3_call(g1, w1p, W2)
    sums, cnts = _p4_reduce(h_slabs.reshape(9 * NPAD, WIDTH), g1, cb_pool)

    tot = sums[:, :B, :].sum(axis=0)
    cnt = cnts[:, :B].sum(axis=0).astype(jnp.float32)
    return tot / jnp.maximum(cnt, 1.0)[:, None]


# speedup vs baseline: 15.8060x; 1.3412x over previous
import functools

import jax
import jax.numpy as jnp
from jax import lax
from jax.experimental import pallas as pl
from jax.experimental.pallas import tpu as pltpu
from jax.experimental.pallas import tpu_sc as plsc

B = 4
IMG_H = 512
IMG_W = 512
N = 200000
IN_CH = 3
WIDTH = 64
HW = IMG_H * IMG_W
SZ = B * HW

NC = 2
NS = 16
NW = NC * NS
LANES = 16

NPAD = 204800
PER_TILE = NPAD // NW

IMG_CH = 4096
IMG_REPS = 17
IMGW = NS * IMG_REPS * IMG_CH
JUNK_SLOT = SZ
EMPTY_SLOT = SZ + 8

OFFSETS = tuple((dy, dx) for dy in (-1, 0, 1) for dx in (-1, 0, 1))

_mesh = plsc.VectorSubcoreMesh(core_axis_name="c", subcore_axis_name="s")
_cp = pltpu.CompilerParams(use_tc_tiling_on_sc=False, needs_layout_passes=False)


def _worker_id():
    return lax.axis_index("s") * NC + lax.axis_index("c")


P1_CW = 1600


@functools.partial(
    pl.kernel,
    out_type=jax.ShapeDtypeStruct((IMGW, 16), jnp.float32),
    mesh=_mesh,
    compiler_params=_cp,
    scratch_types=[
        pltpu.VMEM((IMG_CH, 16), jnp.float32),
        pltpu.VMEM((P1_CW,), jnp.int32),
        pltpu.VMEM((P1_CW,), jnp.int32),
        pltpu.VMEM((P1_CW, 16), jnp.float32),
        pltpu.VMEM((P1_CW, 16), jnp.float32),
        pltpu.SemaphoreType.DMA,
        pltpu.SemaphoreType.DMA,
        pltpu.SemaphoreType.DMA,
    ],
)
def _p1_build_img(pat_hbm, flat_hbm, rows_hbm, img_hbm, fillb, addr0, addr1,
                  src0, src1, fsem, isem, ssem):
    addrb = (addr0, addr1)
    srcb = (src0, src1)
    c = lax.axis_index("c")
    s = lax.axis_index("s")

    @pl.when(c == 0)
    def _():
        pltpu.async_copy(pat_hbm, fillb, fsem).wait()
        for r in range(IMG_REPS):
            pltpu.async_copy(
                fillb, img_hbm.at[pl.ds((s * IMG_REPS + r) * IMG_CH, IMG_CH)],
                fsem)
        for r in range(IMG_REPS):
            pltpu.make_async_copy(
                fillb, img_hbm.at[pl.ds(s * IMG_CH, IMG_CH)], fsem).wait()

    plsc.subcore_barrier()

    @pl.when(c == 0)
    def _():
        per = NPAD // NS
        nch = per // P1_CW

        def load(t, p):
            base = s * per + t * P1_CW
            pltpu.async_copy(flat_hbm.at[pl.ds(base, P1_CW)], addrb[p],
                             isem)
            pltpu.async_copy(rows_hbm.at[pl.ds(base, P1_CW)], srcb[p],
                             isem)

        def wait_load(p):
            pltpu.make_async_copy(flat_hbm.at[pl.ds(0, P1_CW)], addrb[p],
                                  isem).wait()
            pltpu.make_async_copy(rows_hbm.at[pl.ds(0, P1_CW)], srcb[p],
                                  isem).wait()

        def wait_scat(p):
            pltpu.make_async_copy(srcb[p], img_hbm.at[addrb[p]],
                                  ssem).wait()

        @pl.loop(0, nch)
        def _(t):
            load(t, 0)
            wait_load(0)
            pltpu.async_copy(srcb[0], img_hbm.at[addrb[0]], ssem)
            wait_scat(0)


P2_CW = 640
P2_NCH = PER_TILE // P2_CW


@functools.partial(
    pl.kernel,
    out_type=jax.ShapeDtypeStruct((9, NPAD, 16), jnp.float32),
    mesh=_mesh,
    compiler_params=_cp,
    scratch_types=(
        [pltpu.VMEM((2, P2_CW), jnp.int32) for _ in range(2)]
        + [pltpu.VMEM((P2_CW,), jnp.int32) for _ in range(9)]
        + [pltpu.VMEM((P2_CW, 16), jnp.float32) for _ in range(9)]
        + [pltpu.SemaphoreType.DMA, pltpu.SemaphoreType.DMA,
           pltpu.SemaphoreType.DMA, pltpu.SemaphoreType.DMA]
    ),
)
def _p2_gather(flat_hbm, vmask_hbm, img_hbm, g1_hbm, *scr):
    fb, mb = scr[0:2]
    _nf = scr[2:11]
    nfb = [(_nf[k], _nf[k]) for k in range(9)]
    _gb = scr[11:20]
    gb = [(_gb[k], _gb[k]) for k in range(9)]
    isem, gsem0, gsem1, wsem = scr[20:24]
    gsem = (gsem0, gsem1)
    wid = _worker_id()
    base0 = wid * PER_TILE

    def load_inputs(t, p):
        base = base0 + t * P2_CW
        pltpu.async_copy(flat_hbm.at[pl.ds(base, P2_CW)], fb.at[p], isem)
        pltpu.async_copy(vmask_hbm.at[pl.ds(base, P2_CW)], mb.at[p], isem)

    def wait_inputs(p):
        pltpu.make_async_copy(flat_hbm.at[pl.ds(0, P2_CW)], fb.at[p],
                              isem).wait()
        pltpu.make_async_copy(vmask_hbm.at[pl.ds(0, P2_CW)], mb.at[p],
                              isem).wait()

    def compute_nflat(p):
        for k, (dy, dx) in enumerate(OFFSETS):
            d = dy * IMG_W + dx

            @pl.loop(0, P2_CW, step=LANES)
            def _(j, _d=d, _k=k):
                f16 = fb[p, pl.ds(j, LANES)]
                m16 = mb[p, pl.ds(j, LANES)]
                ok = ((m16 >> _k) & 1) == 1
                nfb[_k][p][pl.ds(j, LANES)] = jnp.where(ok, f16 + _d,
                                                        EMPTY_SLOT)

    def fire_gathers(p):
        for k in range(9):
            pltpu.async_copy(img_hbm.at[nfb[k][p]], gb[k][p], gsem[p])

    def wait_gathers(p):
        for k in range(9):
            pltpu.make_async_copy(img_hbm.at[nfb[k][p]], gb[k][p],
                                  gsem[p]).wait()

    def fire_writes(t, p):
        base = base0 + t * P2_CW
        for k in range(9):
            pltpu.async_copy(gb[k][p],
                             g1_hbm.at[k].at[pl.ds(base, P2_CW)], wsem)

    def wait_writes(p):
        for k in range(9):
            pltpu.make_async_copy(gb[k][p],
                                  g1_hbm.at[k].at[pl.ds(0, P2_CW)],
                                  wsem).wait()

    @pl.loop(0, P2_NCH)
    def _(t):
        load_inputs(t, 0)
        wait_inputs(0)
        compute_nflat(0)
        fire_gathers(0)
        wait_gathers(0)
        fire_writes(t, 0)
        wait_writes(0)


P3_CT = 2048


def _p3_body(g1_ref, w1_ref, w2_ref, h_ref):
    acc = jnp.zeros((P3_CT, WIDTH), jnp.float32)
    for k in range(9):
        acc = acc + jnp.dot(g1_ref[k], w1_ref[k],
                            preferred_element_type=jnp.float32)
    h = jnp.maximum(acc, 0.0)
    for k in range(9):
        h_ref[k] = jnp.dot(h, w2_ref[k], preferred_element_type=jnp.float32)


_p3_call = pl.pallas_call(
    _p3_body,
    grid=(NPAD // P3_CT,),
    in_specs=[
        pl.BlockSpec((9, P3_CT, 16), lambda i: (0, i, 0)),
        pl.BlockSpec((9, 16, WIDTH), lambda i: (0, 0, 0)),
        pl.BlockSpec((9, WIDTH, WIDTH), lambda i: (0, 0, 0)),
    ],
    out_specs=pl.BlockSpec((9, P3_CT, WIDTH), lambda i: (0, i, 0)),
    out_shape=jax.ShapeDtypeStruct((9, NPAD, WIDTH), jnp.float32),
)


P4_CW = 128
P4_NCH = PER_TILE // P4_CW


@functools.partial(
    pl.kernel,
    out_type=(
        jax.ShapeDtypeStruct((NW, 16, WIDTH), jnp.float32),
        jax.ShapeDtypeStruct((NW, 16), jnp.int32),
    ),
    mesh=_mesh,
    compiler_params=_cp,
    scratch_types=(
        [pltpu.VMEM((P4_CW, 16), jnp.float32) for _ in range(9)]
        + [pltpu.VMEM((P4_CW,), jnp.int32) for _ in range(9)]
        + [pltpu.VMEM((P4_CW, WIDTH), jnp.float32) for _ in range(9)]
        + [
            pltpu.VMEM((2, P4_CW), jnp.int32),
            pltpu.VMEM((16, WIDTH), jnp.float32),
            pltpu.VMEM((16,), jnp.int32),
            pltpu.SemaphoreType.DMA,
            pltpu.SemaphoreType.DMA,
            pltpu.SemaphoreType.DMA,
        ]
    ),
)
def _p4_reduce(h_hbm, g1_hbm, cb_hbm, sums_hbm, cnt_hbm, *scr):
    _gg = scr[0:9]
    gg = [(_gg[k], _gg[k]) for k in range(9)]
    _ib = scr[9:18]
    idxb = [(_ib[k], _ib[k]) for k in range(9)]
    _hb = scr[18:27]
    hb = [(_hb[k], _hb[k]) for k in range(9)]
    bb, accb, cntb, isem, gsem0, gsem1 = scr[27:33]
    gsem = (gsem0, gsem1)
    wid = _worker_id()
    base0 = wid * PER_TILE

    for r in range(16):
        for cg in range(WIDTH // LANES):
            accb[r, pl.ds(cg * LANES, LANES)] = jnp.zeros((LANES,),
                                                          jnp.float32)
    cntb[...] = jnp.zeros((16,), jnp.int32)

    def load_rows(t, p):
        base = base0 + t * P4_CW
        for k in range(9):
            pltpu.async_copy(g1_hbm.at[k].at[pl.ds(base, P4_CW)],
                             gg[k][p], isem)
        pltpu.async_copy(cb_hbm.at[pl.ds(base, P4_CW)], bb.at[p], isem)

    def wait_rows(p):
        for k in range(9):
            pltpu.make_async_copy(g1_hbm.at[k].at[pl.ds(0, P4_CW)],
                                  gg[k][p], isem).wait()
        pltpu.make_async_copy(cb_hbm.at[pl.ds(0, P4_CW)], bb.at[p],
                              isem).wait()

    def extract_idx(p):
        rid = lax.iota(jnp.int32, LANES)
        col4 = jnp.full((LANES,), 4, jnp.int32)
        for k in range(9):
            for grp in range(P4_CW // LANES):
                v = plsc.load_gather(gg[k][p],
                                     [rid + grp * LANES, col4])
                idxb[k][p][pl.ds(grp * LANES, LANES)] = (
                    v.astype(jnp.int32) + k * NPAD)

    def fire_gathers(p):
        for k in range(9):
            pltpu.async_copy(h_hbm.at[idxb[k][p]], hb[k][p], gsem[p])

    def wait_gathers(p):
        for k in range(9):
            pltpu.make_async_copy(h_hbm.at[idxb[k][p]], hb[k][p],
                                  gsem[p]).wait()

    def compute(p):
        @pl.loop(0, P4_CW, step=LANES)
        def _(j):
            b16 = bb[p, pl.ds(j, LANES)]
            plsc.addupdate_scatter(cntb, [b16],
                                   jnp.full((LANES,), 1, jnp.int32))
            for ri in range(LANES):
                brow = b16[ri]
                r = j + ri
                for cg in range(WIDTH // LANES):
                    sl = pl.ds(cg * LANES, LANES)
                    v01 = hb[0][p][r, sl] + hb[1][p][r, sl]
                    v23 = hb[2][p][r, sl] + hb[3][p][r, sl]
                    v45 = hb[4][p][r, sl] + hb[5][p][r, sl]
                    v67 = hb[6][p][r, sl] + hb[7][p][r, sl]
                    v = ((v01 + v23) + (v45 + v67)) + hb[8][p][r, sl]
                    v = jnp.maximum(v, 0.0)
                    accb[brow, sl] = accb[brow, sl] + v

    @pl.loop(0, P4_NCH)
    def _(t):
        load_rows(t, 0)
        wait_rows(0)
        extract_idx(0)
        fire_gathers(0)
        wait_gathers(0)
        compute(0)

    pltpu.sync_copy(accb, sums_hbm.at[wid])
    pltpu.sync_copy(cntb, cnt_hbm.at[wid])


def kernel(coords, feats, W1, W2):
    ar = jnp.arange(NPAD, dtype=jnp.int32)
    live = ar < N

    cb = jnp.zeros((NPAD,), jnp.int32).at[:N].set(coords[:, 0])
    cy = jnp.zeros((NPAD,), jnp.int32).at[:N].set(coords[:, 1])
    cx = jnp.zeros((NPAD,), jnp.int32).at[:N].set(coords[:, 2])
    flat = cb * HW + cy * IMG_W + cx
    flat = jnp.where(live, flat, JUNK_SLOT)
    cb_pool = jnp.where(live, cb, 8)

    vmask = jnp.zeros((NPAD,), jnp.int32)
    for k, (dy, dx) in enumerate(OFFSETS):
        ok = ((cy + dy >= 0) & (cy + dy < IMG_H)
              & (cx + dx >= 0) & (cx + dx < IMG_W) & live)
        vmask = vmask | (ok.astype(jnp.int32) << k)

    rows = jnp.zeros((NPAD, 16), jnp.float32)
    rows = rows.at[:N, :IN_CH].set(feats)
    rows = rows.at[:, 4].set(ar.astype(jnp.float32))

    w1p = jnp.zeros((9, 16, WIDTH), jnp.float32).at[:, :IN_CH, :].set(W1)

    pat = jnp.zeros((IMG_CH, 16), jnp.float32).at[:, 4].set(float(N))

    cimg = _p1_build_img(pat, flat, rows)
    g1 = _p2_gather(flat, vmask, cimg)
    h_slabs = _p3_call(g1, w1p, W2)
    sums, cnts = _p4_reduce(h_slabs.reshape(9 * NPAD, WIDTH), g1, cb_pool)

    tot = sums[:, :B, :].sum(axis=0)
    cnt = cnts[:, :B].sum(axis=0).astype(jnp.float32)
    return tot / jnp.maximum(cnt, 1.0)[:, None]
